# baseline (speedup 1.0000x reference)
import functools

import jax
import jax.numpy as jnp
from jax import lax
from jax.experimental import pallas as pl
from jax.experimental.pallas import tpu as pltpu
from jax.experimental.pallas import tpu_sc as plsc

_B = 16384
_A = 6
_H = 256
_D = 256
_K = 8192

_NH = 2
_BH = _B // _NH
_BB = 4096
_NB = _BH // _BB
_CK = 1024
_NC = _K // _CK

_NW = 32
_BPW = _BH // _NW
_GCH = 64
_NGC = _BPW // _GCH


def _enc_vq_body(act_ref, w1_ref, b1_ref, w2_ref, b2_ref, w3_ref, b3_ref,
                 ebf_ref, idx_ref, vqs_ref):
    f32 = jnp.float32
    cdims = (((1,), (1,)), ((), ()))
    i = pl.program_id(0)
    x = act_ref[...]
    x = jnp.maximum(
        lax.dot_general(x, w1_ref[...], cdims, preferred_element_type=f32)
        + b1_ref[...], 0.0)
    x = jnp.maximum(
        lax.dot_general(x, w2_ref[...], cdims, preferred_element_type=f32)
        + b2_ref[...], 0.0)
    enc = (lax.dot_general(x, w3_ref[...], cdims, preferred_element_type=f32)
           + b3_ref[...])

    encm2 = (enc * -2.0).astype(jnp.bfloat16)
    ones_row = jnp.ones((1, _D), dtype=f32)
    io10 = lax.broadcasted_iota(jnp.int32, (_BB, _CK), 1)
    maskhi = jnp.int32(~0x3FF)
    best = jnp.full((_BB, 1), jnp.inf, dtype=f32)
    bestc = jnp.zeros((_BB, 1), dtype=jnp.int32)
    for c in range(_NC):
        ec = ebf_ref[c * _CK:(c + 1) * _CK, :]
        ecf = ec.astype(f32)
        n2 = lax.dot_general(ones_row, ecf * ecf, cdims,
                             preferred_element_type=f32)
        dots = lax.dot_general(encm2, ec, cdims,
                               preferred_element_type=f32)
        s = dots + n2
        m = lax.bitcast_convert_type(s, jnp.int32)
        packed = lax.bitcast_convert_type((m & maskhi) | io10, f32)
        cm = jnp.min(packed, axis=1, keepdims=True)
        upd = cm < best
        best = jnp.where(upd, cm, best)
        bestc = jnp.where(upd, c, bestc)
    bbits = lax.bitcast_convert_type(best, jnp.int32)
    idx_ref[0] = bestc * _CK + (bbits & 0x3FF)
    sstar = lax.bitcast_convert_type(bbits & maskhi, f32)

    en2 = jnp.sum(enc * enc, axis=1, keepdims=True)
    pv = jnp.sum(en2 + sstar, axis=0, keepdims=True)

    @pl.when(i == 0)
    def _():
        vqs_ref[...] = pv

    @pl.when(i != 0)
    def _():
        vqs_ref[...] += pv


def _dec_loss_body(q_ref, act_ref, w4_ref, b4_ref, w5_ref, b5_ref,
                   w6_ref, b6_ref, rs_ref):
    f32 = jnp.float32
    cdims = (((1,), (1,)), ((), ()))
    i = pl.program_id(0)
    q = q_ref[...]
    h = jnp.maximum(
        lax.dot_general(q, w4_ref[...], cdims, preferred_element_type=f32)
        + b4_ref[...], 0.0)
    h = jnp.maximum(
        lax.dot_general(h, w5_ref[...], cdims, preferred_element_type=f32)
        + b5_ref[...], 0.0)
    r = jnp.tanh(
        lax.dot_general(h, w6_ref[...], cdims, preferred_element_type=f32)
        + b6_ref[...])
    dr = r - act_ref[...]
    pr = jnp.sum(jnp.sum(dr * dr, axis=1, keepdims=True), axis=0,
                 keepdims=True)

    @pl.when(i == 0)
    def _():
        rs_ref[...] = pr

    @pl.when(i != 0)
    def _():
        rs_ref[...] += pr


def _sc_gather(e, idx):
    mesh = plsc.VectorSubcoreMesh(core_axis_name="c", subcore_axis_name="s")

    @functools.partial(
        pl.kernel, mesh=mesh,
        out_type=jax.ShapeDtypeStruct((_BH, _D), jnp.float32),
        scratch_types=[
            pltpu.VMEM((_BPW,), jnp.int32),
            pltpu.VMEM((_NGC, _GCH, _D), jnp.float32),
            pltpu.SemaphoreType.DMA((_NGC,)),
            pltpu.SemaphoreType.DMA,
        ],
    )
    def gk(e_hbm, idx_hbm, out_hbm, idx_v, rows_v, gsem, wsem):
        wid = lax.axis_index("s") * 2 + lax.axis_index("c")
        base = wid * _BPW
        pltpu.sync_copy(idx_hbm.at[pl.ds(base, _BPW)], idx_v)
        gathers = []
        for c in range(_NGC):
            gathers.append(pltpu.async_copy(
                e_hbm.at[idx_v.at[pl.ds(c * _GCH, _GCH)]],
                rows_v.at[c], gsem.at[c]))
        writes = []
        for c in range(_NGC):
            gathers[c].wait()
            writes.append(pltpu.async_copy(
                rows_v.at[c],
                out_hbm.at[pl.ds(base + c * _GCH, _GCH)], wsem))
        for w in writes:
            w.wait()

    return gk(e, idx)


def _enc_vq(action_h, W1, b1r, W2, b2r, W3, b3r, ebf):
    return pl.pallas_call(
        _enc_vq_body,
        grid=(_NB,),
        in_specs=[
            pl.BlockSpec((_BB, _A), lambda i: (i, 0)),
            pl.BlockSpec((_H, _A), lambda i: (0, 0)),
            pl.BlockSpec((1, _H), lambda i: (0, 0)),
            pl.BlockSpec((_H, _H), lambda i: (0, 0)),
            pl.BlockSpec((1, _H), lambda i: (0, 0)),
            pl.BlockSpec((_D, _H), lambda i: (0, 0)),
            pl.BlockSpec((1, _D), lambda i: (0, 0)),
            pl.BlockSpec((_K, _D), lambda i: (0, 0)),
        ],
        out_specs=[
            pl.BlockSpec((1, _BB, 1), lambda i: (i, 0, 0)),
            pl.BlockSpec((1, 1), lambda i: (0, 0)),
        ],
        out_shape=[
            jax.ShapeDtypeStruct((_NB, _BB, 1), jnp.int32),
            jax.ShapeDtypeStruct((1, 1), jnp.float32),
        ],
    )(action_h, W1, b1r, W2, b2r, W3, b3r, ebf)


def _dec_loss(q, action_h, W4, b4r, W5, b5r, W6, b6r):
    return pl.pallas_call(
        _dec_loss_body,
        grid=(_NB,),
        in_specs=[
            pl.BlockSpec((_BB, _D), lambda i: (i, 0)),
            pl.BlockSpec((_BB, _A), lambda i: (i, 0)),
            pl.BlockSpec((_H, _D), lambda i: (0, 0)),
            pl.BlockSpec((1, _H), lambda i: (0, 0)),
            pl.BlockSpec((_H, _H), lambda i: (0, 0)),
            pl.BlockSpec((1, _H), lambda i: (0, 0)),
            pl.BlockSpec((_A, _H), lambda i: (0, 0)),
            pl.BlockSpec((1, _A), lambda i: (0, 0)),
        ],
        out_specs=pl.BlockSpec((1, 1), lambda i: (0, 0)),
        out_shape=jax.ShapeDtypeStruct((1, 1), jnp.float32),
    )(q, action_h, W4, b4r, W5, b5r, W6, b6r)


def kernel(action, W1, b1, W2, b2, W3, b3, E, W4, b4, W5, b5, W6, b6):
    b1r = b1.reshape(1, _H)
    b2r = b2.reshape(1, _H)
    b3r = b3.reshape(1, _D)
    b4r = b4.reshape(1, _H)
    b5r = b5.reshape(1, _H)
    b6r = b6.reshape(1, _A)
    ebf = E.astype(jnp.bfloat16)

    idxs, vq_parts = [], []
    for h in range(_NH):
        a_h = lax.slice_in_dim(action, h * _BH, (h + 1) * _BH, axis=0)
        idx3, vqs = _enc_vq(a_h, W1, b1r, W2, b2r, W3, b3r, ebf)
        idxs.append(idx3.reshape(_BH))
        vq_parts.append(vqs[0, 0])

    qs = [_sc_gather(E, idxs[h]) for h in range(_NH)]

    rs_parts = []
    for h in range(_NH):
        a_h = lax.slice_in_dim(action, h * _BH, (h + 1) * _BH, axis=0)
        rs = _dec_loss(qs[h], a_h, W4, b4r, W5, b5r, W6, b6r)
        rs_parts.append(rs[0, 0])

    rsum = rs_parts[0] + rs_parts[1]
    vqsum = vq_parts[0] + vq_parts[1]
    return rsum / (_B * _A) + 1.25 * vqsum / (_B * _D)


# speedup vs baseline: 1.0794x; 1.0794x over previous
import functools

import jax
import jax.numpy as jnp
from jax import lax
from jax.experimental import pallas as pl
from jax.experimental.pallas import tpu as pltpu
from jax.experimental.pallas import tpu_sc as plsc

_B = 16384
_A = 6
_H = 256
_D = 256
_K = 8192

_NH = 2
_BH = _B // _NH
_BB = 2048
_NB = _BH // _BB
_CK = 1024
_NC = _K // _CK

_NW = 32
_BPW = _BH // _NW
_GCH = 64
_NGC = _BPW // _GCH


def _enc_vq_body(act_ref, w1_ref, b1_ref, w2_ref, b2_ref, w3_ref, b3_ref,
                 ebf_ref, idx_ref, vqs_ref):
    f32 = jnp.float32
    cdims = (((1,), (1,)), ((), ()))
    i = pl.program_id(0)
    x = act_ref[...]
    x = jnp.maximum(
        lax.dot_general(x, w1_ref[...], cdims, preferred_element_type=f32)
        + b1_ref[...], 0.0)
    x = jnp.maximum(
        lax.dot_general(x, w2_ref[...], cdims, preferred_element_type=f32)
        + b2_ref[...], 0.0)
    enc = (lax.dot_general(x, w3_ref[...], cdims, preferred_element_type=f32)
           + b3_ref[...])

    encm2 = (enc * -2.0).astype(jnp.bfloat16)
    ones_row = jnp.ones((1, _D), dtype=f32)
    io10 = lax.broadcasted_iota(jnp.int32, (_BB, _CK), 1)
    maskhi = jnp.int32(~0x3FF)
    best = jnp.full((_BB, 1), jnp.inf, dtype=f32)
    bestc = jnp.zeros((_BB, 1), dtype=jnp.int32)
    for c in range(_NC):
        ec = ebf_ref[c * _CK:(c + 1) * _CK, :]
        ecf = ec.astype(f32)
        n2 = lax.dot_general(ones_row, ecf * ecf, cdims,
                             preferred_element_type=f32)
        dots = lax.dot_general(encm2, ec, cdims,
                               preferred_element_type=f32)
        s = dots + n2
        m = lax.bitcast_convert_type(s, jnp.int32)
        packed = lax.bitcast_convert_type((m & maskhi) | io10, f32)
        cm = jnp.min(packed, axis=1, keepdims=True)
        upd = cm < best
        best = jnp.where(upd, cm, best)
        bestc = jnp.where(upd, c, bestc)
    bbits = lax.bitcast_convert_type(best, jnp.int32)
    idx_ref[0] = bestc * _CK + (bbits & 0x3FF)
    sstar = lax.bitcast_convert_type(bbits & maskhi, f32)

    en2 = jnp.sum(enc * enc, axis=1, keepdims=True)
    pv = jnp.sum(en2 + sstar, axis=0, keepdims=True)

    @pl.when(i == 0)
    def _():
        vqs_ref[...] = pv

    @pl.when(i != 0)
    def _():
        vqs_ref[...] += pv


def _dec_loss_body(q_ref, act_ref, w4_ref, b4_ref, w5_ref, b5_ref,
                   w6_ref, b6_ref, rs_ref):
    f32 = jnp.float32
    cdims = (((1,), (1,)), ((), ()))
    i = pl.program_id(0)
    q = q_ref[...]
    h = jnp.maximum(
        lax.dot_general(q, w4_ref[...], cdims, preferred_element_type=f32)
        + b4_ref[...], 0.0)
    h = jnp.maximum(
        lax.dot_general(h, w5_ref[...], cdims, preferred_element_type=f32)
        + b5_ref[...], 0.0)
    r = jnp.tanh(
        lax.dot_general(h, w6_ref[...], cdims, preferred_element_type=f32)
        + b6_ref[...])
    dr = r - act_ref[...]
    pr = jnp.sum(jnp.sum(dr * dr, axis=1, keepdims=True), axis=0,
                 keepdims=True)

    @pl.when(i == 0)
    def _():
        rs_ref[...] = pr

    @pl.when(i != 0)
    def _():
        rs_ref[...] += pr


def _sc_gather(e, idx):
    mesh = plsc.VectorSubcoreMesh(core_axis_name="c", subcore_axis_name="s")

    @functools.partial(
        pl.kernel, mesh=mesh,
        out_type=jax.ShapeDtypeStruct((_BH, _D), jnp.float32),
        scratch_types=[
            pltpu.VMEM((_BPW,), jnp.int32),
            pltpu.VMEM((_NGC, _GCH, _D), jnp.float32),
            pltpu.SemaphoreType.DMA((_NGC,)),
            pltpu.SemaphoreType.DMA,
        ],
    )
    def gk(e_hbm, idx_hbm, out_hbm, idx_v, rows_v, gsem, wsem):
        wid = lax.axis_index("s") * 2 + lax.axis_index("c")
        base = wid * _BPW
        pltpu.sync_copy(idx_hbm.at[pl.ds(base, _BPW)], idx_v)
        gathers = []
        for c in range(_NGC):
            gathers.append(pltpu.async_copy(
                e_hbm.at[idx_v.at[pl.ds(c * _GCH, _GCH)]],
                rows_v.at[c], gsem.at[c]))
        writes = []
        for c in range(_NGC):
            gathers[c].wait()
            writes.append(pltpu.async_copy(
                rows_v.at[c],
                out_hbm.at[pl.ds(base + c * _GCH, _GCH)], wsem))
        for w in writes:
            w.wait()

    return gk(e, idx)


def _enc_vq(action_h, W1, b1r, W2, b2r, W3, b3r, ebf):
    return pl.pallas_call(
        _enc_vq_body,
        grid=(_NB,),
        in_specs=[
            pl.BlockSpec((_BB, _A), lambda i: (i, 0)),
            pl.BlockSpec((_H, _A), lambda i: (0, 0)),
            pl.BlockSpec((1, _H), lambda i: (0, 0)),
            pl.BlockSpec((_H, _H), lambda i: (0, 0)),
            pl.BlockSpec((1, _H), lambda i: (0, 0)),
            pl.BlockSpec((_D, _H), lambda i: (0, 0)),
            pl.BlockSpec((1, _D), lambda i: (0, 0)),
            pl.BlockSpec((_K, _D), lambda i: (0, 0)),
        ],
        out_specs=[
            pl.BlockSpec((1, _BB, 1), lambda i: (i, 0, 0)),
            pl.BlockSpec((1, 1), lambda i: (0, 0)),
        ],
        out_shape=[
            jax.ShapeDtypeStruct((_NB, _BB, 1), jnp.int32),
            jax.ShapeDtypeStruct((1, 1), jnp.float32),
        ],
    )(action_h, W1, b1r, W2, b2r, W3, b3r, ebf)


def _dec_loss(q, action_h, W4, b4r, W5, b5r, W6, b6r):
    return pl.pallas_call(
        _dec_loss_body,
        grid=(_NB,),
        in_specs=[
            pl.BlockSpec((_BB, _D), lambda i: (i, 0)),
            pl.BlockSpec((_BB, _A), lambda i: (i, 0)),
            pl.BlockSpec((_H, _D), lambda i: (0, 0)),
            pl.BlockSpec((1, _H), lambda i: (0, 0)),
            pl.BlockSpec((_H, _H), lambda i: (0, 0)),
            pl.BlockSpec((1, _H), lambda i: (0, 0)),
            pl.BlockSpec((_A, _H), lambda i: (0, 0)),
            pl.BlockSpec((1, _A), lambda i: (0, 0)),
        ],
        out_specs=pl.BlockSpec((1, 1), lambda i: (0, 0)),
        out_shape=jax.ShapeDtypeStruct((1, 1), jnp.float32),
    )(q, action_h, W4, b4r, W5, b5r, W6, b6r)


def kernel(action, W1, b1, W2, b2, W3, b3, E, W4, b4, W5, b5, W6, b6):
    b1r = b1.reshape(1, _H)
    b2r = b2.reshape(1, _H)
    b3r = b3.reshape(1, _D)
    b4r = b4.reshape(1, _H)
    b5r = b5.reshape(1, _H)
    b6r = b6.reshape(1, _A)
    ebf = E.astype(jnp.bfloat16)

    idxs, vq_parts = [], []
    for h in range(_NH):
        a_h = lax.slice_in_dim(action, h * _BH, (h + 1) * _BH, axis=0)
        idx3, vqs = _enc_vq(a_h, W1, b1r, W2, b2r, W3, b3r, ebf)
        idxs.append(idx3.reshape(_BH))
        vq_parts.append(vqs[0, 0])

    qs = [_sc_gather(E, idxs[h]) for h in range(_NH)]

    rs_parts = []
    for h in range(_NH):
        a_h = lax.slice_in_dim(action, h * _BH, (h + 1) * _BH, axis=0)
        rs = _dec_loss(qs[h], a_h, W4, b4r, W5, b5r, W6, b6r)
        rs_parts.append(rs[0, 0])

    rsum = rs_parts[0] + rs_parts[1]
    vqsum = vq_parts[0] + vq_parts[1]
    return rsum / (_B * _A) + 1.25 * vqsum / (_B * _D)
